# Initial kernel scaffold; baseline (speedup 1.0000x reference)
import jax
import jax.numpy as jnp
from jax.experimental import pallas as pl


def kernel(x, edge_index, edge_weight, W1, b1, W2, b2):
    raise NotImplementedError("write your pallas kernel here")


# speedup vs baseline: 23.7709x; 23.7709x over previous
import functools

import jax
import jax.numpy as jnp
from jax import lax
from jax.experimental import pallas as pl
from jax.experimental.pallas import tpu as pltpu
from jax.experimental.pallas import tpu_sc as plsc

NEG_SLOPE = 0.01
CH = 128
NC = 2
NS = 16
NW = NC * NS

_mesh = functools.partial(
    plsc.VectorSubcoreMesh, core_axis_name="c", subcore_axis_name="s")


def _make_k_deg(n, cpt):
    @functools.partial(
        pl.kernel,
        out_type=jax.ShapeDtypeStruct((NC, n), jnp.float32),
        mesh=_mesh(),
        scratch_types=[
            pltpu.VMEM((cpt, CH), jnp.int32),
            pltpu.VMEM((cpt, CH), jnp.float32),
            pltpu.VMEM_SHARED((n,), jnp.float32),
        ],
    )
    def k_deg(col_hbm, ew_hbm, zn_hbm, deg_out, col_v, ew_v, deg_sh):
        cid = lax.axis_index("c")
        sid = lax.axis_index("s")
        wid = sid * NC + cid

        @pl.when(sid == 0)
        def _():
            pltpu.sync_copy(zn_hbm, deg_sh)

        pltpu.sync_copy(col_hbm.at[wid], col_v)
        pltpu.sync_copy(ew_hbm.at[wid], ew_v)
        plsc.subcore_barrier()

        def body(ci, carry):
            pltpu.sync_copy(ew_v.at[ci], deg_sh.at[col_v.at[ci]], add=True)
            return carry

        lax.fori_loop(0, cpt, body, 0)
        plsc.subcore_barrier()

        @pl.when(sid == 0)
        def _():
            pltpu.sync_copy(deg_sh, deg_out.at[cid])

    return k_deg


def _make_k_prop(n, h, cpt):
    nsplit = 10
    rows_per_tile = n // nsplit
    assert n % nsplit == 0 and rows_per_tile % 8 == 0

    @functools.partial(
        pl.kernel,
        out_type=jax.ShapeDtypeStruct((NC, n, h), jnp.float32),
        mesh=_mesh(),
        scratch_types=[
            pltpu.VMEM((cpt, CH), jnp.int32),
            pltpu.VMEM((cpt, CH), jnp.int32),
            pltpu.VMEM((cpt, CH), jnp.float32),
            pltpu.VMEM((CH, h), jnp.float32),
            pltpu.VMEM_SHARED((n, h), jnp.float32),
            pltpu.SemaphoreType.DMA,
        ],
        compiler_params=pltpu.CompilerParams(use_tc_tiling_on_sc=False),
    )
    def k_prop(g_hbm, row_hbm, col_hbm, ew_hbm, zacc_hbm, out_hbm,
               row_v, col_v, ew_v, rows_v, acc_sh, sem):
        cid = lax.axis_index("c")
        sid = lax.axis_index("s")
        wid = sid * NC + cid

        @pl.when(sid == 0)
        def _():
            pltpu.sync_copy(zacc_hbm, acc_sh)

        pltpu.sync_copy(row_hbm.at[wid], row_v)
        pltpu.sync_copy(col_hbm.at[wid], col_v)
        pltpu.sync_copy(ew_hbm.at[wid], ew_v)
        plsc.subcore_barrier()

        def chunk(ci, carry):
            pltpu.async_copy(g_hbm.at[row_v.at[ci]], rows_v, sem).wait()

            def scale(q, c2):
                ew16 = ew_v[ci, pl.ds(q * 16, 16)]
                for j in range(16):
                    s = ew16[j]
                    r = q * 16 + j
                    for k in range(h // 16):
                        sl = pl.ds(k * 16, 16)
                        rows_v[r, sl] = rows_v[r, sl] * s
                return c2

            lax.fori_loop(0, CH // 16, scale, 0)
            pltpu.sync_copy(rows_v, acc_sh.at[col_v.at[ci]], add=True)
            return carry

        lax.fori_loop(0, cpt, chunk, 0)
        plsc.subcore_barrier()

        @pl.when(sid < nsplit)
        def _():
            base = sid * rows_per_tile
            pltpu.sync_copy(acc_sh.at[pl.ds(base, rows_per_tile)],
                            out_hbm.at[cid, pl.ds(base, rows_per_tile)])

    return k_prop


def _tc1_body(deg_ref, x_ref, w1_ref, dis_ref, g1_ref):
    deg = deg_ref[0] + deg_ref[1] + 1.0
    dis = lax.rsqrt(deg)
    dis_ref[...] = dis
    hmat = jnp.dot(x_ref[...], w1_ref[...], preferred_element_type=jnp.float32)
    g1_ref[...] = hmat * dis


def _tc2_body(acc_ref, g1_ref, dis_ref, b1_ref, w2_ref, g2_ref):
    dis = dis_ref[...]
    pre = dis * (acc_ref[0] + acc_ref[1] + g1_ref[...]) + b1_ref[...]
    z = jnp.where(pre > 0, pre, NEG_SLOPE * pre)
    h2 = jnp.dot(z, w2_ref[...], preferred_element_type=jnp.float32)
    g2_ref[...] = h2 * dis


def _tc3_body(acc_ref, g2_ref, dis_ref, b2_ref, out_ref):
    dis = dis_ref[...]
    out_ref[...] = dis * (acc_ref[0] + acc_ref[1] + g2_ref[...]) + b2_ref[...]


def kernel(x, edge_index, edge_weight, W1, b1, W2, b2):
    n, d = x.shape
    h1 = W1.shape[1]
    h2 = W2.shape[1]
    e = edge_index.shape[1]

    cpt = -(-e // (NW * CH))
    e_pad = NW * cpt * CH
    pad = e_pad - e

    row_p = jnp.pad(edge_index[0], (0, pad)).reshape(NW, cpt, CH)
    col_p = jnp.pad(edge_index[1], (0, pad)).reshape(NW, cpt, CH)
    ew_p = jnp.pad(edge_weight, (0, pad)).reshape(NW, cpt, CH)
    zn = jnp.zeros((n,), jnp.float32)
    zacc1 = jnp.zeros((n, h1), jnp.float32)

    k_deg = _make_k_deg(n, cpt)
    k_prop1 = _make_k_prop(n, h1, cpt)
    k_prop2 = k_prop1 if h2 == h1 else _make_k_prop(n, h2, cpt)

    deg = k_deg(col_p, ew_p, zn)

    k_tc1 = pl.pallas_call(
        _tc1_body,
        out_shape=(jax.ShapeDtypeStruct((n, 1), jnp.float32),
                   jax.ShapeDtypeStruct((n, h1), jnp.float32)),
    )
    dis, g1 = k_tc1(deg.reshape(NC, n, 1), x, W1)

    acc1 = k_prop1(g1, row_p, col_p, ew_p, zacc1)

    k_tc2 = pl.pallas_call(
        _tc2_body,
        out_shape=jax.ShapeDtypeStruct((n, h2), jnp.float32),
    )
    g2 = k_tc2(acc1, g1, dis, b1.reshape(1, h1), W2)

    zacc2 = zacc1 if h2 == h1 else jnp.zeros((n, h2), jnp.float32)
    acc2 = k_prop2(g2, row_p, col_p, ew_p, zacc2)

    k_tc3 = pl.pallas_call(
        _tc3_body,
        out_shape=jax.ShapeDtypeStruct((n, h2), jnp.float32),
    )
    return k_tc3(acc2, g2, dis, b2.reshape(1, h2))


# speedup vs baseline: 24.1122x; 1.0144x over previous
import functools

import jax
import jax.numpy as jnp
from jax import lax
from jax.experimental import pallas as pl
from jax.experimental.pallas import tpu as pltpu
from jax.experimental.pallas import tpu_sc as plsc

NEG_SLOPE = 0.01
CH = 128
NC = 2
NS = 16
NW = NC * NS

_mesh = functools.partial(
    plsc.VectorSubcoreMesh, core_axis_name="c", subcore_axis_name="s")


def _make_k_deg(n, cpt):
    @functools.partial(
        pl.kernel,
        out_type=jax.ShapeDtypeStruct((NC, n), jnp.float32),
        mesh=_mesh(),
        scratch_types=[
            pltpu.VMEM((cpt, CH), jnp.int32),
            pltpu.VMEM((cpt, CH), jnp.float32),
            pltpu.VMEM_SHARED((n,), jnp.float32),
        ],
    )
    def k_deg(col_hbm, ew_hbm, zn_hbm, deg_out, col_v, ew_v, deg_sh):
        cid = lax.axis_index("c")
        sid = lax.axis_index("s")
        wid = sid * NC + cid

        @pl.when(sid == 0)
        def _():
            pltpu.sync_copy(zn_hbm, deg_sh)

        pltpu.sync_copy(col_hbm.at[wid], col_v)
        pltpu.sync_copy(ew_hbm.at[wid], ew_v)
        plsc.subcore_barrier()

        def body(ci, carry):
            pltpu.sync_copy(ew_v.at[ci], deg_sh.at[col_v.at[ci]], add=True)
            return carry

        lax.fori_loop(0, cpt, body, 0)
        plsc.subcore_barrier()

        @pl.when(sid == 0)
        def _():
            pltpu.sync_copy(deg_sh, deg_out.at[cid])

    return k_deg


def _make_k_prop(n, h, cpt):
    nsplit = 10
    rows_per_tile = n // nsplit
    assert n % nsplit == 0 and rows_per_tile % 8 == 0

    @functools.partial(
        pl.kernel,
        out_type=jax.ShapeDtypeStruct((NC, n, h), jnp.float32),
        mesh=_mesh(),
        scratch_types=[
            pltpu.VMEM((cpt, CH), jnp.int32),
            pltpu.VMEM((cpt, CH), jnp.int32),
            pltpu.VMEM((cpt, CH), jnp.float32),
            pltpu.VMEM((CH, h), jnp.float32),
            pltpu.VMEM((CH, h), jnp.float32),
            pltpu.VMEM_SHARED((n, h), jnp.float32),
            pltpu.SemaphoreType.DMA,
            pltpu.SemaphoreType.DMA,
        ],
        compiler_params=pltpu.CompilerParams(use_tc_tiling_on_sc=False),
    )
    def k_prop(g_hbm, row_hbm, col_hbm, ew_hbm, zacc_hbm, out_hbm,
               row_v, col_v, ew_v, rows_a, rows_b, acc_sh, sem_a, sem_b):
        cid = lax.axis_index("c")
        sid = lax.axis_index("s")
        wid = sid * NC + cid

        @pl.when(sid == 0)
        def _():
            pltpu.sync_copy(zacc_hbm, acc_sh)

        pltpu.sync_copy(row_hbm.at[wid], row_v)
        pltpu.sync_copy(col_hbm.at[wid], col_v)
        pltpu.sync_copy(ew_hbm.at[wid], ew_v)
        plsc.subcore_barrier()

        def scale_and_scatter(ci, rows_v):
            def scale(q, c2):
                ew16 = ew_v[ci, pl.ds(q * 16, 16)]
                for j in range(16):
                    s = ew16[j]
                    r = q * 16 + j
                    for k in range(h // 16):
                        sl = pl.ds(k * 16, 16)
                        rows_v[r, sl] = rows_v[r, sl] * s
                return c2

            lax.fori_loop(0, CH // 16, scale, 0)
            pltpu.sync_copy(rows_v, acc_sh.at[col_v.at[ci]], add=True)

        niter = cpt // 2
        pltpu.async_copy(g_hbm.at[row_v.at[0]], rows_a, sem_a)

        def body(i, carry):
            c0 = 2 * i
            pltpu.make_async_copy(g_hbm.at[row_v.at[c0]], rows_a, sem_a).wait()
            pltpu.async_copy(g_hbm.at[row_v.at[c0 + 1]], rows_b, sem_b)
            scale_and_scatter(c0, rows_a)

            @pl.when(i < niter - 1)
            def _():
                pltpu.async_copy(g_hbm.at[row_v.at[c0 + 2]], rows_a, sem_a)

            pltpu.make_async_copy(g_hbm.at[row_v.at[c0 + 1]], rows_b, sem_b).wait()
            scale_and_scatter(c0 + 1, rows_b)
            return carry

        lax.fori_loop(0, niter, body, 0)
        plsc.subcore_barrier()

        @pl.when(sid < nsplit)
        def _():
            base = sid * rows_per_tile
            pltpu.sync_copy(acc_sh.at[pl.ds(base, rows_per_tile)],
                            out_hbm.at[cid, pl.ds(base, rows_per_tile)])

    return k_prop


def _tc1_body(deg_ref, x_ref, w1_ref, dis_ref, g1_ref):
    deg = deg_ref[0] + deg_ref[1] + 1.0
    dis = lax.rsqrt(deg)
    dis_ref[...] = dis
    hmat = jnp.dot(x_ref[...], w1_ref[...], preferred_element_type=jnp.float32)
    g1_ref[...] = hmat * dis


def _tc2_body(acc_ref, g1_ref, dis_ref, b1_ref, w2_ref, g2_ref):
    dis = dis_ref[...]
    pre = dis * (acc_ref[0] + acc_ref[1] + g1_ref[...]) + b1_ref[...]
    z = jnp.where(pre > 0, pre, NEG_SLOPE * pre)
    h2 = jnp.dot(z, w2_ref[...], preferred_element_type=jnp.float32)
    g2_ref[...] = h2 * dis


def _tc3_body(acc_ref, g2_ref, dis_ref, b2_ref, out_ref):
    dis = dis_ref[...]
    out_ref[...] = dis * (acc_ref[0] + acc_ref[1] + g2_ref[...]) + b2_ref[...]


def kernel(x, edge_index, edge_weight, W1, b1, W2, b2):
    n, d = x.shape
    h1 = W1.shape[1]
    h2 = W2.shape[1]
    e = edge_index.shape[1]

    cpt = -(-e // (NW * CH))
    cpt += cpt % 2
    e_pad = NW * cpt * CH
    pad = e_pad - e

    row_p = jnp.pad(edge_index[0], (0, pad)).reshape(NW, cpt, CH)
    col_p = jnp.pad(edge_index[1], (0, pad)).reshape(NW, cpt, CH)
    ew_p = jnp.pad(edge_weight, (0, pad)).reshape(NW, cpt, CH)
    zn = jnp.zeros((n,), jnp.float32)
    zacc1 = jnp.zeros((n, h1), jnp.float32)

    k_deg = _make_k_deg(n, cpt)
    k_prop1 = _make_k_prop(n, h1, cpt)
    k_prop2 = k_prop1 if h2 == h1 else _make_k_prop(n, h2, cpt)

    deg = k_deg(col_p, ew_p, zn)

    k_tc1 = pl.pallas_call(
        _tc1_body,
        out_shape=(jax.ShapeDtypeStruct((n, 1), jnp.float32),
                   jax.ShapeDtypeStruct((n, h1), jnp.float32)),
    )
    dis, g1 = k_tc1(deg.reshape(NC, n, 1), x, W1)

    acc1 = k_prop1(g1, row_p, col_p, ew_p, zacc1)

    k_tc2 = pl.pallas_call(
        _tc2_body,
        out_shape=jax.ShapeDtypeStruct((n, h2), jnp.float32),
    )
    g2 = k_tc2(acc1, g1, dis, b1.reshape(1, h1), W2)

    zacc2 = zacc1 if h2 == h1 else jnp.zeros((n, h2), jnp.float32)
    acc2 = k_prop2(g2, row_p, col_p, ew_p, zacc2)

    k_tc3 = pl.pallas_call(
        _tc3_body,
        out_shape=jax.ShapeDtypeStruct((n, h2), jnp.float32),
    )
    return k_tc3(acc2, g2, dis, b2.reshape(1, h2))


# speedup vs baseline: 41.7841x; 1.7329x over previous
import functools

import jax
import jax.numpy as jnp
from jax import lax
from jax.experimental import pallas as pl
from jax.experimental.pallas import tpu as pltpu
from jax.experimental.pallas import tpu_sc as plsc

NEG_SLOPE = 0.01
CH = 128
NC = 2
NS = 16
NW = NC * NS

_mesh = functools.partial(
    plsc.VectorSubcoreMesh, core_axis_name="c", subcore_axis_name="s")


def _make_k_deg(n, cpt):
    @functools.partial(
        pl.kernel,
        out_type=jax.ShapeDtypeStruct((NC, n), jnp.float32),
        mesh=_mesh(),
        scratch_types=[
            pltpu.VMEM((cpt, CH), jnp.int32),
            pltpu.VMEM((cpt, CH), jnp.float32),
            pltpu.VMEM_SHARED((n,), jnp.float32),
        ],
    )
    def k_deg(col_hbm, ew_hbm, zn_hbm, deg_out, col_v, ew_v, deg_sh):
        cid = lax.axis_index("c")
        sid = lax.axis_index("s")
        wid = sid * NC + cid

        @pl.when(sid == 0)
        def _():
            pltpu.sync_copy(zn_hbm, deg_sh)

        pltpu.sync_copy(col_hbm.at[wid], col_v)
        pltpu.sync_copy(ew_hbm.at[wid], ew_v)
        plsc.subcore_barrier()

        def body(ci, carry):
            pltpu.sync_copy(ew_v.at[ci], deg_sh.at[col_v.at[ci]], add=True)
            return carry

        lax.fori_loop(0, cpt, body, 0)
        plsc.subcore_barrier()

        @pl.when(sid == 0)
        def _():
            pltpu.sync_copy(deg_sh, deg_out.at[cid])

    return k_deg


def _make_k_prop(n, h, cpt):
    nsplit = 10
    rows_per_tile = n // nsplit
    assert n % nsplit == 0 and rows_per_tile % 8 == 0

    @functools.partial(
        pl.kernel,
        out_type=jax.ShapeDtypeStruct((NC, n, h), jnp.float32),
        mesh=_mesh(),
        scratch_types=[
            pltpu.VMEM((cpt, CH), jnp.int32),
            pltpu.VMEM((cpt, CH), jnp.int32),
            pltpu.VMEM((cpt, CH), jnp.float32),
            pltpu.VMEM((CH, h), jnp.float32),
            pltpu.VMEM((CH, h), jnp.float32),
            pltpu.VMEM_SHARED((n, h), jnp.float32),
            pltpu.VMEM_SHARED((n, h), jnp.float32),
            pltpu.SemaphoreType.DMA,
            pltpu.SemaphoreType.DMA,
        ],
        compiler_params=pltpu.CompilerParams(use_tc_tiling_on_sc=False),
    )
    def k_prop(g_hbm, row_hbm, col_hbm, ew_hbm, zacc_hbm, out_hbm,
               row_v, col_v, ew_v, rows_a, rows_b, acc_sh, g_sh,
               sem_a, sem_b):
        cid = lax.axis_index("c")
        sid = lax.axis_index("s")
        wid = sid * NC + cid

        @pl.when(sid < nsplit)
        def _():
            sbase = sid * rows_per_tile
            sl = pl.ds(sbase, rows_per_tile)
            pltpu.sync_copy(zacc_hbm.at[sl], acc_sh.at[sl])
            pltpu.sync_copy(g_hbm.at[sl], g_sh.at[sl])

        pltpu.sync_copy(row_hbm.at[wid], row_v)
        pltpu.sync_copy(col_hbm.at[wid], col_v)
        pltpu.sync_copy(ew_hbm.at[wid], ew_v)
        plsc.subcore_barrier()

        def scale_and_scatter(ci, rows_v):
            def scale(q, c2):
                ew16 = ew_v[ci, pl.ds(q * 16, 16)]
                for j in range(16):
                    s = ew16[j]
                    r = q * 16 + j
                    for k in range(h // 16):
                        sl = pl.ds(k * 16, 16)
                        rows_v[r, sl] = rows_v[r, sl] * s
                return c2

            lax.fori_loop(0, CH // 16, scale, 0)
            pltpu.sync_copy(rows_v, acc_sh.at[col_v.at[ci]], add=True)

        niter = cpt // 2
        pltpu.async_copy(g_sh.at[row_v.at[0]], rows_a, sem_a)

        def body(i, carry):
            c0 = 2 * i
            pltpu.make_async_copy(g_sh.at[row_v.at[c0]], rows_a, sem_a).wait()
            pltpu.async_copy(g_sh.at[row_v.at[c0 + 1]], rows_b, sem_b)
            scale_and_scatter(c0, rows_a)

            @pl.when(i < niter - 1)
            def _():
                pltpu.async_copy(g_sh.at[row_v.at[c0 + 2]], rows_a, sem_a)

            pltpu.make_async_copy(g_sh.at[row_v.at[c0 + 1]], rows_b, sem_b).wait()
            scale_and_scatter(c0 + 1, rows_b)
            return carry

        lax.fori_loop(0, niter, body, 0)
        plsc.subcore_barrier()

        @pl.when(sid < nsplit)
        def _():
            base = sid * rows_per_tile
            pltpu.sync_copy(acc_sh.at[pl.ds(base, rows_per_tile)],
                            out_hbm.at[cid, pl.ds(base, rows_per_tile)])

    return k_prop


def _tc1_body(deg_ref, x_ref, w1_ref, dis_ref, g1_ref):
    deg = deg_ref[0] + deg_ref[1] + 1.0
    dis = lax.rsqrt(deg)
    dis_ref[...] = dis
    hmat = jnp.dot(x_ref[...], w1_ref[...], preferred_element_type=jnp.float32)
    g1_ref[...] = hmat * dis


def _tc2_body(acc_ref, g1_ref, dis_ref, b1_ref, w2_ref, g2_ref):
    dis = dis_ref[...]
    pre = dis * (acc_ref[0] + acc_ref[1] + g1_ref[...]) + b1_ref[...]
    z = jnp.where(pre > 0, pre, NEG_SLOPE * pre)
    h2 = jnp.dot(z, w2_ref[...], preferred_element_type=jnp.float32)
    g2_ref[...] = h2 * dis


def _tc3_body(acc_ref, g2_ref, dis_ref, b2_ref, out_ref):
    dis = dis_ref[...]
    out_ref[...] = dis * (acc_ref[0] + acc_ref[1] + g2_ref[...]) + b2_ref[...]


def kernel(x, edge_index, edge_weight, W1, b1, W2, b2):
    n, d = x.shape
    h1 = W1.shape[1]
    h2 = W2.shape[1]
    e = edge_index.shape[1]

    cpt = -(-e // (NW * CH))
    cpt += cpt % 2
    e_pad = NW * cpt * CH
    pad = e_pad - e

    row_p = jnp.pad(edge_index[0], (0, pad)).reshape(NW, cpt, CH)
    col_p = jnp.pad(edge_index[1], (0, pad)).reshape(NW, cpt, CH)
    ew_p = jnp.pad(edge_weight, (0, pad)).reshape(NW, cpt, CH)
    zn = jnp.zeros((n,), jnp.float32)
    zacc1 = jnp.zeros((n, h1), jnp.float32)

    k_deg = _make_k_deg(n, cpt)
    k_prop1 = _make_k_prop(n, h1, cpt)
    k_prop2 = k_prop1 if h2 == h1 else _make_k_prop(n, h2, cpt)

    deg = k_deg(col_p, ew_p, zn)

    k_tc1 = pl.pallas_call(
        _tc1_body,
        out_shape=(jax.ShapeDtypeStruct((n, 1), jnp.float32),
                   jax.ShapeDtypeStruct((n, h1), jnp.float32)),
    )
    dis, g1 = k_tc1(deg.reshape(NC, n, 1), x, W1)

    acc1 = k_prop1(g1, row_p, col_p, ew_p, zacc1)

    k_tc2 = pl.pallas_call(
        _tc2_body,
        out_shape=jax.ShapeDtypeStruct((n, h2), jnp.float32),
    )
    g2 = k_tc2(acc1, g1, dis, b1.reshape(1, h1), W2)

    zacc2 = zacc1 if h2 == h1 else jnp.zeros((n, h2), jnp.float32)
    acc2 = k_prop2(g2, row_p, col_p, ew_p, zacc2)

    k_tc3 = pl.pallas_call(
        _tc3_body,
        out_shape=jax.ShapeDtypeStruct((n, h2), jnp.float32),
    )
    return k_tc3(acc2, g2, dis, b2.reshape(1, h2))


# speedup vs baseline: 44.8646x; 1.0737x over previous
import functools

import jax
import jax.numpy as jnp
from jax import lax
from jax.experimental import pallas as pl
from jax.experimental.pallas import tpu as pltpu
from jax.experimental.pallas import tpu_sc as plsc

NEG_SLOPE = 0.01
CH = 128
NC = 2
NS = 16
NW = NC * NS

_mesh = functools.partial(
    plsc.VectorSubcoreMesh, core_axis_name="c", subcore_axis_name="s")


def _make_k_deg(n, cpt):
    @functools.partial(
        pl.kernel,
        out_type=jax.ShapeDtypeStruct((NC, n), jnp.float32),
        mesh=_mesh(),
        scratch_types=[
            pltpu.VMEM((cpt, CH), jnp.int32),
            pltpu.VMEM((cpt, CH), jnp.float32),
            pltpu.VMEM_SHARED((n,), jnp.float32),
        ],
    )
    def k_deg(col_hbm, ew_hbm, zn_hbm, deg_out, col_v, ew_v, deg_sh):
        cid = lax.axis_index("c")
        sid = lax.axis_index("s")
        wid = sid * NC + cid

        @pl.when(sid == 0)
        def _():
            pltpu.sync_copy(zn_hbm, deg_sh)

        pltpu.sync_copy(col_hbm.at[wid], col_v)
        pltpu.sync_copy(ew_hbm.at[wid], ew_v)
        plsc.subcore_barrier()

        def body(ci, carry):
            pltpu.sync_copy(ew_v.at[ci], deg_sh.at[col_v.at[ci]], add=True)
            return carry

        lax.fori_loop(0, cpt, body, 0)
        plsc.subcore_barrier()

        @pl.when(sid == 0)
        def _():
            pltpu.sync_copy(deg_sh, deg_out.at[cid])

    return k_deg


def _make_k_prop(n, h, cpt):
    nsplit = 10
    rows_per_tile = n // nsplit
    assert n % nsplit == 0 and rows_per_tile % 8 == 0

    @functools.partial(
        pl.kernel,
        out_type=jax.ShapeDtypeStruct((NC, n, h), jnp.float32),
        mesh=_mesh(),
        scratch_types=[
            pltpu.VMEM((cpt, CH), jnp.int32),
            pltpu.VMEM((cpt, CH), jnp.int32),
            pltpu.VMEM((cpt, CH), jnp.float32),
            [pltpu.VMEM((CH, h), jnp.float32)] * 4,
            pltpu.VMEM_SHARED((n, h), jnp.float32),
            pltpu.VMEM_SHARED((n, h), jnp.float32),
            [pltpu.SemaphoreType.DMA] * 4,
            [pltpu.SemaphoreType.DMA] * 4,
        ],
        compiler_params=pltpu.CompilerParams(use_tc_tiling_on_sc=False),
    )
    def k_prop(g_hbm, row_hbm, col_hbm, ew_hbm, zacc_hbm, out_hbm,
               row_v, col_v, ew_v, rows, acc_sh, g_sh, gsem, ssem):
        cid = lax.axis_index("c")
        sid = lax.axis_index("s")
        wid = sid * NC + cid

        @pl.when(sid < nsplit)
        def _():
            sbase = sid * rows_per_tile
            sl = pl.ds(sbase, rows_per_tile)
            pltpu.sync_copy(zacc_hbm.at[sl], acc_sh.at[sl])
            pltpu.sync_copy(g_hbm.at[sl], g_sh.at[sl])

        pltpu.sync_copy(row_hbm.at[wid], row_v)
        pltpu.sync_copy(col_hbm.at[wid], col_v)
        pltpu.sync_copy(ew_hbm.at[wid], ew_v)
        plsc.subcore_barrier()

        def scale(ci, rows_v):
            def scale16(q, c2):
                ew16 = ew_v[ci, pl.ds(q * 16, 16)]
                for j in range(16):
                    s = ew16[j]
                    r = q * 16 + j
                    for k in range(h // 16):
                        sl = pl.ds(k * 16, 16)
                        rows_v[r, sl] = rows_v[r, sl] * s
                return c2

            lax.fori_loop(0, CH // 16, scale16, 0)

        def gather(ci, b):
            pltpu.async_copy(g_sh.at[row_v.at[ci]], rows[b], gsem[b])

        def wait_gather(ci, b):
            pltpu.make_async_copy(g_sh.at[row_v.at[ci]], rows[b], gsem[b]).wait()

        def scatter(ci, b):
            pltpu.async_copy(rows[b], acc_sh.at[col_v.at[ci]], ssem[b],
                             add=True)

        def wait_scatter(ci, b):
            pltpu.make_async_copy(rows[b], acc_sh.at[col_v.at[ci]],
                                  ssem[b]).wait()

        niter = cpt // 4
        gather(0, 0)
        gather(1, 1)

        def body(i, carry):
            for k in range(4):
                c = 4 * i + k
                b = k
                bn = (k + 2) % 4
                wait_gather(c, b)
                scale(c, rows[b])
                scatter(c, b)

                @pl.when(c + 2 < cpt)
                def _():
                    @pl.when(c >= 2)
                    def _():
                        wait_scatter(c - 2, bn)

                    gather(c + 2, bn)

            return carry

        lax.fori_loop(0, niter, body, 0)
        for c in range(cpt - 4, cpt):
            wait_scatter(c, c % 4)
        plsc.subcore_barrier()

        @pl.when(sid < nsplit)
        def _():
            base = sid * rows_per_tile
            pltpu.sync_copy(acc_sh.at[pl.ds(base, rows_per_tile)],
                            out_hbm.at[cid, pl.ds(base, rows_per_tile)])

    return k_prop


def _tc1_body(deg_ref, x_ref, w1_ref, dis_ref, g1_ref):
    deg = deg_ref[0] + deg_ref[1] + 1.0
    dis = lax.rsqrt(deg)
    dis_ref[...] = dis
    hmat = jnp.dot(x_ref[...], w1_ref[...], preferred_element_type=jnp.float32)
    g1_ref[...] = hmat * dis


def _tc2_body(acc_ref, g1_ref, dis_ref, b1_ref, w2_ref, g2_ref):
    dis = dis_ref[...]
    pre = dis * (acc_ref[0] + acc_ref[1] + g1_ref[...]) + b1_ref[...]
    z = jnp.where(pre > 0, pre, NEG_SLOPE * pre)
    h2 = jnp.dot(z, w2_ref[...], preferred_element_type=jnp.float32)
    g2_ref[...] = h2 * dis


def _tc3_body(acc_ref, g2_ref, dis_ref, b2_ref, out_ref):
    dis = dis_ref[...]
    out_ref[...] = dis * (acc_ref[0] + acc_ref[1] + g2_ref[...]) + b2_ref[...]


def kernel(x, edge_index, edge_weight, W1, b1, W2, b2):
    n, d = x.shape
    h1 = W1.shape[1]
    h2 = W2.shape[1]
    e = edge_index.shape[1]

    cpt = -(-e // (NW * CH))
    cpt += cpt % 2
    e_pad = NW * cpt * CH
    pad = e_pad - e

    row_p = jnp.pad(edge_index[0], (0, pad)).reshape(NW, cpt, CH)
    col_p = jnp.pad(edge_index[1], (0, pad)).reshape(NW, cpt, CH)
    ew_p = jnp.pad(edge_weight, (0, pad)).reshape(NW, cpt, CH)
    zn = jnp.zeros((n,), jnp.float32)
    zacc1 = jnp.zeros((n, h1), jnp.float32)

    k_deg = _make_k_deg(n, cpt)
    k_prop1 = _make_k_prop(n, h1, cpt)
    k_prop2 = k_prop1 if h2 == h1 else _make_k_prop(n, h2, cpt)

    deg = k_deg(col_p, ew_p, zn)

    k_tc1 = pl.pallas_call(
        _tc1_body,
        out_shape=(jax.ShapeDtypeStruct((n, 1), jnp.float32),
                   jax.ShapeDtypeStruct((n, h1), jnp.float32)),
    )
    dis, g1 = k_tc1(deg.reshape(NC, n, 1), x, W1)

    acc1 = k_prop1(g1, row_p, col_p, ew_p, zacc1)

    k_tc2 = pl.pallas_call(
        _tc2_body,
        out_shape=jax.ShapeDtypeStruct((n, h2), jnp.float32),
    )
    g2 = k_tc2(acc1, g1, dis, b1.reshape(1, h1), W2)

    zacc2 = zacc1 if h2 == h1 else jnp.zeros((n, h2), jnp.float32)
    acc2 = k_prop2(g2, row_p, col_p, ew_p, zacc2)

    k_tc3 = pl.pallas_call(
        _tc3_body,
        out_shape=jax.ShapeDtypeStruct((n, h2), jnp.float32),
    )
    return k_tc3(acc2, g2, dis, b2.reshape(1, h2))


# speedup vs baseline: 55.2547x; 1.2316x over previous
import functools

import jax
import jax.numpy as jnp
from jax import lax
from jax.experimental import pallas as pl
from jax.experimental.pallas import tpu as pltpu
from jax.experimental.pallas import tpu_sc as plsc

NEG_SLOPE = 0.01
CH = 128
NC = 2
NS = 16
NW = NC * NS


_mesh = functools.partial(
    plsc.VectorSubcoreMesh, core_axis_name="c", subcore_axis_name="s")


def _make_k_deg(np_, h, slab, tail):
    q = 128 // h
    p = np_ // q
    stile = (-(-p // NS) + 7) // 8 * 8
    srem = p - stile * (NS - 1)
    assert 0 < srem <= stile
    nbuf = -(-stile // 16) * 16

    @functools.partial(
        pl.kernel,
        out_type=jax.ShapeDtypeStruct((NC, p, 128), jnp.float32),
        mesh=_mesh(),
        scratch_types=[
            pltpu.VMEM((slab, CH), jnp.int32),
            pltpu.VMEM((slab, CH), jnp.float32),
            pltpu.VMEM((nbuf,), jnp.float32),
            pltpu.VMEM((nbuf, 128), jnp.float32),
            pltpu.VMEM_SHARED((np_,), jnp.float32),
        ],
        compiler_params=pltpu.CompilerParams(use_tc_tiling_on_sc=False),
    )
    def k_deg(col_hbm, ew_hbm, zn_hbm, z2_hbm, deg_out, col_v, ew_v, dloc,
              rep, deg_sh):
        cid = lax.axis_index("c")
        sid = lax.axis_index("s")
        wid = sid * NC + cid

        @pl.when(sid == 0)
        def _():
            pltpu.sync_copy(zn_hbm, deg_sh)

        @pl.when(wid < NW - 1)
        def _():
            pltpu.sync_copy(col_hbm.at[pl.ds(wid * slab, slab)], col_v)
            pltpu.sync_copy(ew_hbm.at[pl.ds(wid * slab, slab)], ew_v)

        @pl.when(wid == NW - 1)
        def _():
            sl = pl.ds((NW - 1) * slab, tail)
            dst = pl.ds(0, tail)
            pltpu.sync_copy(col_hbm.at[sl], col_v.at[dst])
            pltpu.sync_copy(ew_hbm.at[sl], ew_v.at[dst])

        plsc.subcore_barrier()

        def body(ci, carry):
            pltpu.sync_copy(ew_v.at[ci], deg_sh.at[col_v.at[ci]], add=True)
            return carry

        @pl.when(wid < NW - 1)
        def _():
            lax.fori_loop(0, slab, body, 0)

        @pl.when(wid == NW - 1)
        def _():
            lax.fori_loop(0, tail, body, 0)

        plsc.subcore_barrier()

        def replicate(off, cnt):
            pltpu.sync_copy(z2_hbm, rep)
            for i in range(q):
                pltpu.sync_copy(deg_sh.at[pl.ds(i * p + off, cnt)],
                                dloc.at[pl.ds(0, cnt)])

                def rep16(g, carry):
                    d16 = dloc[pl.ds(g * 16, 16)]
                    for j in range(16):
                        s = d16[j]
                        r = g * 16 + j
                        for k in range(h // 16):
                            sl = pl.ds(i * h + k * 16, 16)
                            rep[r, sl] = rep[r, sl] + s
                    return carry

                lax.fori_loop(0, -(-cnt // 16), rep16, 0)
            pltpu.sync_copy(rep.at[pl.ds(0, cnt)],
                            deg_out.at[cid, pl.ds(off, cnt)])

        @pl.when(sid < NS - 1)
        def _():
            replicate(sid * stile, stile)

        @pl.when(sid == NS - 1)
        def _():
            replicate((NS - 1) * stile, srem)

    return k_deg


def _make_k_prop(np_, h, slab, tail):
    q = 128 // h
    p = np_ // q
    stile = (-(-p // NS) + 7) // 8 * 8
    srem = p - stile * (NS - 1)
    assert 0 < srem <= stile

    @functools.partial(
        pl.kernel,
        out_type=jax.ShapeDtypeStruct((NC, p, 128), jnp.float32),
        mesh=_mesh(),
        scratch_types=[
            pltpu.VMEM((slab, CH), jnp.int32),
            pltpu.VMEM((slab, CH), jnp.int32),
            pltpu.VMEM((slab, CH), jnp.float32),
            [pltpu.VMEM((CH, h), jnp.float32)] * 4,
            pltpu.VMEM_SHARED((np_, h), jnp.float32),
            pltpu.VMEM_SHARED((np_, h), jnp.float32),
            [pltpu.SemaphoreType.DMA] * 4,
            [pltpu.SemaphoreType.DMA] * 4,
        ],
        compiler_params=pltpu.CompilerParams(use_tc_tiling_on_sc=False),
    )
    def k_prop(g_hbm, row_hbm, col_hbm, ew_hbm, zacc_hbm, out_hbm,
               row_v, col_v, ew_v, rows, acc_sh, g_sh, gsem, ssem):
        cid = lax.axis_index("c")
        sid = lax.axis_index("s")
        wid = sid * NC + cid

        def stage(off, cnt):
            pltpu.sync_copy(zacc_hbm.at[pl.ds(off * q, cnt * q)],
                            acc_sh.at[pl.ds(off * q, cnt * q)])
            for i in range(q):
                pltpu.sync_copy(g_hbm.at[pl.ds(off, cnt), pl.ds(i * h, h)],
                                g_sh.at[pl.ds(i * p + off, cnt)])

        @pl.when(sid < NS - 1)
        def _():
            stage(sid * stile, stile)

        @pl.when(sid == NS - 1)
        def _():
            stage((NS - 1) * stile, srem)

        @pl.when(wid < NW - 1)
        def _():
            sl = pl.ds(wid * slab, slab)
            pltpu.sync_copy(row_hbm.at[sl], row_v)
            pltpu.sync_copy(col_hbm.at[sl], col_v)
            pltpu.sync_copy(ew_hbm.at[sl], ew_v)

        @pl.when(wid == NW - 1)
        def _():
            sl = pl.ds((NW - 1) * slab, tail)
            dst = pl.ds(0, tail)
            pltpu.sync_copy(row_hbm.at[sl], row_v.at[dst])
            pltpu.sync_copy(col_hbm.at[sl], col_v.at[dst])
            pltpu.sync_copy(ew_hbm.at[sl], ew_v.at[dst])

        plsc.subcore_barrier()

        def scale(ci, rows_v):
            def scale16(q, c2):
                ew16 = ew_v[ci, pl.ds(q * 16, 16)]
                for j in range(16):
                    s = ew16[j]
                    r = q * 16 + j
                    for k in range(h // 16):
                        sl = pl.ds(k * 16, 16)
                        rows_v[r, sl] = rows_v[r, sl] * s
                return c2

            lax.fori_loop(0, CH // 16, scale16, 0)

        def gather(ci, b):
            pltpu.async_copy(g_sh.at[row_v.at[ci]], rows[b], gsem[b])

        def wait_gather(ci, b):
            pltpu.make_async_copy(g_sh.at[row_v.at[ci]], rows[b], gsem[b]).wait()

        def scatter(ci, b):
            pltpu.async_copy(rows[b], acc_sh.at[col_v.at[ci]], ssem[b],
                             add=True)

        def wait_scatter(ci, b):
            pltpu.make_async_copy(rows[b], acc_sh.at[col_v.at[ci]],
                                  ssem[b]).wait()

        def pipeline(nck):
            gather(0, 0)
            gather(1, 1)

            def body(i, carry):
                for k in range(4):
                    c = 4 * i + k
                    b = k
                    bn = (k + 2) % 4
                    wait_gather(c, b)
                    scale(c, rows[b])
                    scatter(c, b)

                    @pl.when(c + 2 < nck)
                    def _():
                        @pl.when(c >= 2)
                        def _():
                            wait_scatter(c - 2, bn)

                        gather(c + 2, bn)

                return carry

            lax.fori_loop(0, nck // 4, body, 0)
            for c in range(nck - 4, nck):
                wait_scatter(c, c % 4)

        @pl.when(wid < NW - 1)
        def _():
            pipeline(slab)

        @pl.when(wid == NW - 1)
        def _():
            pipeline(tail)

        plsc.subcore_barrier()

        def copy_out(off, cnt):
            for i in range(q):
                pltpu.sync_copy(acc_sh.at[pl.ds(i * p + off, cnt)],
                                out_hbm.at[cid, pl.ds(off, cnt),
                                           pl.ds(i * h, h)])

        @pl.when(sid < NS - 1)
        def _():
            copy_out(sid * stile, stile)

        @pl.when(sid == NS - 1)
        def _():
            copy_out((NS - 1) * stile, srem)

    return k_prop


def _tc1_body(n, np_, h, deg4_ref, x_ref, w1_ref, dis4_ref, g1_ref):
    q = 128 // h
    p = np_ // q
    dis4 = lax.rsqrt(deg4_ref[0] + deg4_ref[1] + 1.0)
    dis4_ref[...] = dis4
    x = x_ref[...]
    w1 = w1_ref[...]
    parts = []
    for i in range(q):
        lo = min(i * p, n)
        hi = min(i * p + p, n)
        hb = jnp.dot(x[lo:hi], w1, preferred_element_type=jnp.float32)
        parts.append(jnp.pad(hb, ((0, p - (hi - lo)), (0, 0))))
    h1p = jnp.concatenate(parts, axis=1)
    g1_ref[...] = dis4 * h1p


def _tc2_body(acc_ref, g1_ref, dis4_ref, b1_ref, w2_ref, g2_ref):
    dis4 = dis4_ref[...]
    pre = dis4 * (acc_ref[0] + acc_ref[1] + g1_ref[...]) + b1_ref[...]
    z = jnp.where(pre > 0, pre, NEG_SLOPE * pre)
    h2 = jnp.dot(z, w2_ref[...], preferred_element_type=jnp.float32)
    g2_ref[...] = h2 * dis4


def _tc3_body(acc_ref, g2_ref, dis4_ref, b2_ref, out_ref):
    out_ref[...] = (dis4_ref[...] * (acc_ref[0] + acc_ref[1] + g2_ref[...])
                    + b2_ref[...])


def kernel(x, edge_index, edge_weight, W1, b1, W2, b2):
    n, d = x.shape
    h = W1.shape[1]
    assert W2.shape[1] == h and 128 % h == 0
    e = edge_index.shape[1]
    q = 128 // h

    np_ = -(-n // (8 * q)) * (8 * q)
    p = np_ * h // 128
    assert n % q == 0

    chunks = -(-e // CH)
    slab = max(4, -(-(-(-chunks // NW)) // 4) * 4)
    tail = chunks - (NW - 1) * slab
    row_f = edge_index[0]
    col_f = edge_index[1]
    ew_f = edge_weight
    if e != chunks * CH or tail <= 0 or tail % 4 != 0:
        slab = max(4, -(-chunks // NW))
        slab = -(-slab // 4) * 4
        e_pad = NW * slab * CH
        tail = slab
        chunks = NW * slab
        row_f = jnp.pad(row_f, (0, e_pad - e))
        col_f = jnp.pad(col_f, (0, e_pad - e))
        ew_f = jnp.pad(ew_f, (0, e_pad - e))
    row_f = row_f.reshape(chunks, CH)
    col_f = col_f.reshape(chunks, CH)
    ew_f = ew_f.reshape(chunks, CH)

    zn = jnp.zeros((np_,), jnp.float32)
    zacc = jnp.zeros((np_, h), jnp.float32)
    stile = (-(-p // NS) + 7) // 8 * 8
    nbuf = -(-stile // 16) * 16
    z2 = jnp.zeros((nbuf, 128), jnp.float32)
    b1t = jnp.tile(b1, q).reshape(1, 128)
    b2t = jnp.tile(b2, q).reshape(1, 128)
    w2k = jnp.kron(jnp.eye(q, dtype=jnp.float32), W2)

    k_deg = _make_k_deg(np_, h, slab, tail)
    k_prop = _make_k_prop(np_, h, slab, tail)

    deg4 = k_deg(col_f, ew_f, zn, z2)

    k_tc1 = pl.pallas_call(
        functools.partial(_tc1_body, n, np_, h),
        out_shape=(jax.ShapeDtypeStruct((p, 128), jnp.float32),
                   jax.ShapeDtypeStruct((p, 128), jnp.float32)),
    )
    dis4, g1 = k_tc1(deg4, x, W1)

    acc1 = k_prop(g1, row_f, col_f, ew_f, zacc)

    k_tc2 = pl.pallas_call(
        _tc2_body,
        out_shape=jax.ShapeDtypeStruct((p, 128), jnp.float32),
    )
    g2 = k_tc2(acc1, g1, dis4, b1t, w2k)

    acc2 = k_prop(g2, row_f, col_f, ew_f, zacc)

    k_tc3 = pl.pallas_call(
        _tc3_body,
        out_shape=jax.ShapeDtypeStruct((p, 128), jnp.float32),
    )
    outp = k_tc3(acc2, g2, dis4, b2t)
    out = jnp.concatenate([outp[:, i * h:(i + 1) * h] for i in range(q)],
                          axis=0)
    return out[:n]
